# Initial kernel scaffold; baseline (speedup 1.0000x reference)
import jax
import jax.numpy as jnp
from jax.experimental import pallas as pl


def kernel(inputs, targets, weights, date, dates):
    raise NotImplementedError("write your pallas kernel here")


# speedup vs baseline: 1.0924x; 1.0924x over previous
import jax
import jax.numpy as jnp
from jax.experimental import pallas as pl
from jax.experimental.pallas import tpu as pltpu

_LANES = 128
_SUB = 8
_DAY_SLOTS = 8
_TILE_ROWS = 2048


def _round_up(x, m):
    return ((x + m - 1) // m) * m


def _make_body(alpha, scaling, nblk, nsteps, ndays):
    half = 0.5 * float(scaling)

    def body(x_ref, t_ref, w_ref, d_ref, dates_ref, out_ref, acc_ref):
        j = pl.program_id(0)

        @pl.when(j == 0)
        def _init():
            acc_ref[...] = jnp.zeros_like(acc_ref)

        x = x_ref[...]
        t = t_ref[...]
        w = w_ref[...]
        d = d_ref[...]
        sig0 = 0.5 + 0.5 * jnp.tanh(half * x[0])
        sig1 = 0.5 + 0.5 * jnp.tanh(half * x[1])
        v = w * (t[0] * sig0 + t[1] * sig1)
        vr = v.reshape(nblk, _SUB, _LANES)
        dr = d.reshape(nblk, _SUB, _LANES)
        for k in range(ndays):
            day = dates_ref[k]
            acc_ref[k] += jnp.sum(jnp.where(dr == day, vr, 0.0), axis=0)

        @pl.when(j == nsteps - 1)
        def _fin():
            q = jnp.sum(acc_ref[...], axis=1)
            pi = jnp.sum(q, axis=1, keepdims=True)
            s = jnp.sum(pi, axis=0, keepdims=True)
            ssq = jnp.sum(pi * pi, axis=0, keepdims=True)
            out_ref[...] = (-float(alpha) * s * jnp.maximum(s, 0.0)
                            / ssq / float(ndays))

    return body


def kernel(inputs, targets, weights, date, dates):
    alpha = 1.0
    scaling = 12.0
    n_targets = int(inputs.shape[-1])
    w1 = weights.reshape(-1)
    d1 = date.reshape(-1).astype(jnp.float32)
    n = int(w1.shape[0])
    ndays = int(dates.shape[0])
    dates = dates.astype(jnp.float32)

    x2 = inputs.reshape(-1).reshape(n_targets, n)
    t2 = targets.reshape(-1).reshape(n_targets, n)

    block_elems = _TILE_ROWS * _LANES
    n_pad = _round_up(n, block_elems)
    pad = n_pad - n
    rows = n_pad // _LANES
    nsteps = rows // _TILE_ROWS

    x3 = jnp.pad(x2, ((0, 0), (0, pad))).reshape(n_targets, rows, _LANES)
    t3 = jnp.pad(t2, ((0, 0), (0, pad))).reshape(n_targets, rows, _LANES)
    w3 = jnp.pad(w1, (0, pad)).reshape(rows, _LANES)
    d3 = jnp.pad(d1, (0, pad)).reshape(rows, _LANES)

    nblk = _TILE_ROWS // _SUB
    body = _make_body(alpha, scaling, nblk, nsteps, ndays)

    out = pl.pallas_call(
        body,
        out_shape=jax.ShapeDtypeStruct((1, 1), jnp.float32),
        grid=(nsteps,),
        in_specs=[
            pl.BlockSpec((n_targets, _TILE_ROWS, _LANES), lambda j: (0, j, 0)),
            pl.BlockSpec((n_targets, _TILE_ROWS, _LANES), lambda j: (0, j, 0)),
            pl.BlockSpec((_TILE_ROWS, _LANES), lambda j: (j, 0)),
            pl.BlockSpec((_TILE_ROWS, _LANES), lambda j: (j, 0)),
            pl.BlockSpec(memory_space=pltpu.MemorySpace.SMEM),
        ],
        out_specs=pl.BlockSpec((1, 1), lambda j: (0, 0)),
        scratch_shapes=[pltpu.VMEM((_DAY_SLOTS, _SUB, _LANES), jnp.float32)],
        compiler_params=pltpu.CompilerParams(
            dimension_semantics=("arbitrary",),
            vmem_limit_bytes=64 << 20,
        ),
    )(x3, t3, w3, d3, dates)
    return out[0, 0]
